# Initial kernel scaffold; baseline (speedup 1.0000x reference)
import jax
import jax.numpy as jnp
from jax.experimental import pallas as pl


def kernel(concepts, relations, W1, b1, W2, b2):
    raise NotImplementedError("write your pallas kernel here")


# speedup vs baseline: 7.4821x; 7.4821x over previous
import functools

import jax
import jax.numpy as jnp
from jax import lax
from jax.experimental import pallas as pl
from jax.experimental.pallas import tpu as pltpu
from jax.experimental.pallas import tpu_sc as plsc

NC = 2
NS = 16
NW = NC * NS
B = 128
SEG = 40
DW = 128


def _sc_degree(n_pad, n_chunks):
    zr = n_pad // NS
    n_segs = n_chunks // SEG
    mesh = plsc.VectorSubcoreMesh(core_axis_name="c", subcore_axis_name="s")

    @functools.partial(
        pl.kernel,
        mesh=mesh,
        out_type=jax.ShapeDtypeStruct((NC, n_pad, DW), jnp.float32),
        scratch_types=[
            pltpu.VMEM((SEG, B), jnp.int32),
            pltpu.VMEM((B, DW), jnp.float32),
            pltpu.VMEM_SHARED((n_pad, DW), jnp.float32),
        ],
    )
    def deg_kernel(dst_hbm, z_hbm, ones_hbm, out_hbm, idx_v, ones_v, acc):
        c = lax.axis_index("c")
        s = lax.axis_index("s")
        wid = s * NC + c
        pltpu.sync_copy(ones_hbm, ones_v)
        pltpu.sync_copy(z_hbm, acc.at[pl.ds(s * zr, zr)])
        plsc.subcore_barrier()

        def body(j, carry):
            pltpu.sync_copy(ones_v, acc.at[idx_v.at[j]], add=True)
            return carry

        for seg in range(n_segs):
            pltpu.sync_copy(dst_hbm.at[wid, pl.ds(seg * SEG, SEG)], idx_v)
            lax.fori_loop(0, SEG, body, 0)
        plsc.subcore_barrier()
        pltpu.sync_copy(acc.at[pl.ds(s * zr, zr)],
                        out_hbm.at[c, pl.ds(s * zr, zr)])

    return deg_kernel


def _sc_aggregate(n_pad, d, n_chunks):
    zr = n_pad // NS
    n_segs = n_chunks // SEG
    mesh = plsc.VectorSubcoreMesh(core_axis_name="c", subcore_axis_name="s")

    @functools.partial(
        pl.kernel,
        mesh=mesh,
        out_type=jax.ShapeDtypeStruct((NC, n_pad, d), jnp.float32),
        scratch_types=[
            pltpu.VMEM((SEG, B), jnp.int32),
            pltpu.VMEM((SEG, B), jnp.int32),
            pltpu.VMEM((B, d), jnp.float32),
            pltpu.VMEM((B, d), jnp.float32),
            pltpu.VMEM_SHARED((n_pad, d), jnp.float32),
            pltpu.SemaphoreType.DMA,
        ],
    )
    def agg_kernel(hs_hbm, src_hbm, dst_hbm, z_hbm, out_hbm,
                   src_v, dst_v, rows0, rows1, acc, sem):
        c = lax.axis_index("c")
        s = lax.axis_index("s")
        wid = s * NC + c
        pltpu.sync_copy(z_hbm, acc.at[pl.ds(s * zr, zr)])
        plsc.subcore_barrier()

        def body(i, carry):
            j0 = 2 * i
            pltpu.make_async_copy(hs_hbm.at[src_v.at[j0]], rows0, sem).wait()
            pltpu.async_copy(hs_hbm.at[src_v.at[j0 + 1]], rows1, sem)
            pltpu.sync_copy(rows0, acc.at[dst_v.at[j0]], add=True)
            pltpu.make_async_copy(
                hs_hbm.at[src_v.at[j0 + 1]], rows1, sem).wait()

            @pl.when(j0 + 2 < SEG)
            def _():
                pltpu.async_copy(hs_hbm.at[src_v.at[j0 + 2]], rows0, sem)

            pltpu.sync_copy(rows1, acc.at[dst_v.at[j0 + 1]], add=True)
            return carry

        for seg in range(n_segs):
            pltpu.sync_copy(src_hbm.at[wid, pl.ds(seg * SEG, SEG)], src_v)
            pltpu.sync_copy(dst_hbm.at[wid, pl.ds(seg * SEG, SEG)], dst_v)
            pltpu.async_copy(hs_hbm.at[src_v.at[0]], rows0, sem)
            lax.fori_loop(0, SEG // 2, body, 0)
        plsc.subcore_barrier()
        pltpu.sync_copy(acc.at[pl.ds(s * zr, zr)],
                        out_hbm.at[c, pl.ds(s * zr, zr)])

    return agg_kernel


def _dinv_block(degp_ref):
    deg = degp_ref[0][:, 0:1] + degp_ref[1][:, 0:1] + 1.0
    return lax.rsqrt(deg)


def _tc_in_body(x_ref, w_ref, degp_ref, o_ref):
    h = jnp.dot(x_ref[...], w_ref[...], preferred_element_type=jnp.float32)
    o_ref[...] = h * _dinv_block(degp_ref)


def _tc_mid_body(g_ref, hs_ref, degp_ref, b_ref, w_ref, o_ref):
    dinv = _dinv_block(degp_ref)
    x1 = jnp.maximum((g_ref[0] + g_ref[1] + hs_ref[...]) * dinv + b_ref[...],
                     0.0)
    h2 = jnp.dot(x1, w_ref[...], preferred_element_type=jnp.float32)
    o_ref[...] = h2 * dinv


def _tc_final_body(g_ref, hs_ref, degp_ref, b_ref, x0_ref, o_ref):
    dinv = _dinv_block(degp_ref)
    o_ref[...] = (x0_ref[...]
                  + (g_ref[0] + g_ref[1] + hs_ref[...]) * dinv + b_ref[...])


def kernel(concepts, relations, W1, b1, W2, b2):
    n, d = concepts.shape
    e = relations.shape[1]
    assert n % NS == 0, n
    n_pad = ((n + NS * 8) // (NS * 8)) * (NS * 8)
    n_chunks = -(-e // (NW * B * SEG)) * SEG
    e_pad = n_chunks * NW * B

    rel = relations.astype(jnp.int32)
    pad = e_pad - e
    src = jnp.concatenate([rel[0], jnp.zeros((pad,), jnp.int32)])
    dst = jnp.concatenate([rel[1], jnp.full((pad,), n, jnp.int32)])
    src3 = src.reshape(NW, n_chunks, B)
    dst3 = dst.reshape(NW, n_chunks, B)
    ones_rows = jnp.ones((B, DW), jnp.float32)
    zrows = jnp.zeros((n_pad // NS, d), jnp.float32)
    b1r = b1.reshape(1, d)
    b2r = b2.reshape(1, d)

    degp = _sc_degree(n_pad, n_chunks)(dst3, zrows, ones_rows)

    r = 1000 if n % 1000 == 0 else (n // NS)
    grid = (n // r,)
    row_spec = pl.BlockSpec((r, d), lambda i: (i, 0))
    w_spec = pl.BlockSpec((d, d), lambda i: (0, 0))
    deg_spec = pl.BlockSpec((NC, r, DW), lambda i: (0, i, 0))
    g_spec = pl.BlockSpec((NC, r, d), lambda i: (0, i, 0))
    b_spec = pl.BlockSpec((1, d), lambda i: (0, 0))
    out_sds = jax.ShapeDtypeStruct((n, d), jnp.float32)

    hs1 = pl.pallas_call(
        _tc_in_body, grid=grid,
        in_specs=[row_spec, w_spec, deg_spec],
        out_specs=row_spec, out_shape=out_sds,
    )(concepts, W1, degp)

    agg = _sc_aggregate(n_pad, d, n_chunks)
    g1 = agg(hs1, src3, dst3, zrows)

    hs2 = pl.pallas_call(
        _tc_mid_body, grid=grid,
        in_specs=[g_spec, row_spec, deg_spec, b_spec, w_spec],
        out_specs=row_spec, out_shape=out_sds,
    )(g1, hs1, degp, b1r, W2)

    g2 = agg(hs2, src3, dst3, zrows)

    out = pl.pallas_call(
        _tc_final_body, grid=grid,
        in_specs=[g_spec, row_spec, deg_spec, b_spec, row_spec],
        out_specs=row_spec, out_shape=out_sds,
    )(g2, hs2, degp, b2r, concepts)
    return out


# speedup vs baseline: 8.7195x; 1.1654x over previous
import functools

import jax
import jax.numpy as jnp
from jax import lax
from jax.experimental import pallas as pl
from jax.experimental.pallas import tpu as pltpu
from jax.experimental.pallas import tpu_sc as plsc

NC = 2
NS = 16
NW = NC * NS
B = 128
SEG = 40
DW = 128


def _sc_degree(n_pad, n_chunks):
    zr = n_pad // NS
    n_segs = n_chunks // SEG
    mesh = plsc.VectorSubcoreMesh(core_axis_name="c", subcore_axis_name="s")

    @functools.partial(
        pl.kernel,
        mesh=mesh,
        out_type=jax.ShapeDtypeStruct((NC, n_pad, DW), jnp.float32),
        scratch_types=[
            pltpu.VMEM((SEG, B), jnp.int32),
            pltpu.VMEM((B, DW), jnp.float32),
            pltpu.VMEM_SHARED((n_pad, DW), jnp.float32),
        ],
    )
    def deg_kernel(dst_hbm, z_hbm, ones_hbm, out_hbm, idx_v, ones_v, acc):
        c = lax.axis_index("c")
        s = lax.axis_index("s")
        wid = s * NC + c
        pltpu.sync_copy(ones_hbm, ones_v)
        pltpu.sync_copy(z_hbm, acc.at[pl.ds(s * zr, zr)])
        plsc.subcore_barrier()

        def body(j, carry):
            pltpu.sync_copy(ones_v, acc.at[idx_v.at[j]], add=True)
            return carry

        for seg in range(n_segs):
            pltpu.sync_copy(dst_hbm.at[wid, pl.ds(seg * SEG, SEG)], idx_v)
            lax.fori_loop(0, SEG, body, 0)
        plsc.subcore_barrier()
        pltpu.sync_copy(acc.at[pl.ds(s * zr, zr)],
                        out_hbm.at[c, pl.ds(s * zr, zr)])

    return deg_kernel


def _sc_aggregate(n_pad, d, segs0, segs1):
    zr = n_pad // NS
    mesh = plsc.VectorSubcoreMesh(core_axis_name="c", subcore_axis_name="s")
    spt = SEG // 8

    @functools.partial(
        pl.kernel,
        mesh=mesh,
        out_type=jax.ShapeDtypeStruct((NC, n_pad, d), jnp.float32),
        scratch_types=[
            pltpu.VMEM((SEG, B), jnp.int32),
            pltpu.VMEM((SEG, B), jnp.int32),
            pltpu.VMEM((B, d), jnp.float32),
            pltpu.VMEM((B, d), jnp.float32),
            pltpu.VMEM_SHARED((n_pad, d), jnp.float32),
            pltpu.SemaphoreType.DMA,
        ],
    )
    def agg_kernel(hs_hbm, src_hbm, dst_hbm, z_hbm, out_hbm,
                   src_v, dst_v, rows0, rows1, acc, sem):
        c = lax.axis_index("c")
        s = lax.axis_index("s")
        base8 = jnp.where(c == 0, s * (segs0 * spt),
                          NS * (segs0 * spt) + s * (segs1 * spt))
        n_segs = jnp.where(c == 0, segs0, segs1)
        pltpu.sync_copy(z_hbm, acc.at[pl.ds(s * zr, zr)])
        plsc.subcore_barrier()

        def body(i, carry):
            j0 = 2 * i
            pltpu.make_async_copy(hs_hbm.at[src_v.at[j0]], rows0, sem).wait()
            pltpu.async_copy(hs_hbm.at[src_v.at[j0 + 1]], rows1, sem)
            pltpu.sync_copy(rows0, acc.at[dst_v.at[j0]], add=True)
            pltpu.make_async_copy(
                hs_hbm.at[src_v.at[j0 + 1]], rows1, sem).wait()

            @pl.when(j0 + 2 < SEG)
            def _():
                pltpu.async_copy(hs_hbm.at[src_v.at[j0 + 2]], rows0, sem)

            pltpu.sync_copy(rows1, acc.at[dst_v.at[j0 + 1]], add=True)
            return carry

        def seg_body(seg, carry):
            off = (base8 + seg * spt) * 8
            pltpu.sync_copy(src_hbm.at[pl.ds(off, SEG)], src_v)
            pltpu.sync_copy(dst_hbm.at[pl.ds(off, SEG)], dst_v)
            pltpu.async_copy(hs_hbm.at[src_v.at[0]], rows0, sem)
            lax.fori_loop(0, SEG // 2, body, 0)
            return carry

        lax.fori_loop(0, n_segs, seg_body, 0)
        plsc.subcore_barrier()
        pltpu.sync_copy(acc.at[pl.ds(s * zr, zr)],
                        out_hbm.at[c, pl.ds(s * zr, zr)])

    return agg_kernel


def _dinv_block(degp_ref):
    deg = degp_ref[0][:, 0:1] + degp_ref[1][:, 0:1] + 1.0
    return lax.rsqrt(deg)


def _tc_in_body(x_ref, w_ref, degp_ref, o_ref):
    h = jnp.dot(x_ref[...], w_ref[...], preferred_element_type=jnp.float32)
    o_ref[...] = h * _dinv_block(degp_ref)


def _tc_mid_body(g_ref, hs_ref, degp_ref, b_ref, w_ref, o_ref):
    dinv = _dinv_block(degp_ref)
    x1 = jnp.maximum((g_ref[0] + g_ref[1] + hs_ref[...]) * dinv + b_ref[...],
                     0.0)
    h2 = jnp.dot(x1, w_ref[...], preferred_element_type=jnp.float32)
    o_ref[...] = h2 * dinv


def _tc_final_body(g_ref, hs_ref, degp_ref, b_ref, x0_ref, o_ref):
    dinv = _dinv_block(degp_ref)
    o_ref[...] = (x0_ref[...]
                  + (g_ref[0] + g_ref[1] + hs_ref[...]) * dinv + b_ref[...])


def kernel(concepts, relations, W1, b1, W2, b2):
    n, d = concepts.shape
    e = relations.shape[1]
    assert n % NS == 0, n
    n_pad = ((n + NS * 8) // (NS * 8)) * (NS * 8)
    rel = relations.astype(jnp.int32)

    n_chunks = -(-e // (NW * B * SEG)) * SEG
    pad = n_chunks * NW * B - e
    dst3 = jnp.concatenate(
        [rel[1], jnp.full((pad,), n, jnp.int32)]).reshape(NW, n_chunks, B)

    s_tot = -(-e // (NS * B * SEG))
    segs0 = max(1, min(s_tot - 1, round(s_tot * 0.75))) if s_tot > 1 else 1
    segs1 = s_tot - segs0
    tot_chunks = NS * s_tot * SEG
    padf = tot_chunks * B - e
    srcf = jnp.concatenate(
        [rel[0], jnp.zeros((padf,), jnp.int32)]).reshape(tot_chunks, B)
    dstf = jnp.concatenate(
        [rel[1], jnp.full((padf,), n, jnp.int32)]).reshape(tot_chunks, B)

    ones_rows = jnp.ones((B, DW), jnp.float32)
    zrows = jnp.zeros((n_pad // NS, d), jnp.float32)
    b1r = b1.reshape(1, d)
    b2r = b2.reshape(1, d)

    degp = _sc_degree(n_pad, n_chunks)(dst3, zrows, ones_rows)

    r = 1000 if n % 1000 == 0 else (n // NS)
    grid = (n // r,)
    row_spec = pl.BlockSpec((r, d), lambda i: (i, 0))
    w_spec = pl.BlockSpec((d, d), lambda i: (0, 0))
    deg_spec = pl.BlockSpec((NC, r, DW), lambda i: (0, i, 0))
    g_spec = pl.BlockSpec((NC, r, d), lambda i: (0, i, 0))
    b_spec = pl.BlockSpec((1, d), lambda i: (0, 0))
    out_sds = jax.ShapeDtypeStruct((n, d), jnp.float32)

    hs1 = pl.pallas_call(
        _tc_in_body, grid=grid,
        in_specs=[row_spec, w_spec, deg_spec],
        out_specs=row_spec, out_shape=out_sds,
    )(concepts, W1, degp)

    agg = _sc_aggregate(n_pad, d, segs0, segs1)
    g1 = agg(hs1, srcf, dstf, zrows)

    hs2 = pl.pallas_call(
        _tc_mid_body, grid=grid,
        in_specs=[g_spec, row_spec, deg_spec, b_spec, w_spec],
        out_specs=row_spec, out_shape=out_sds,
    )(g1, hs1, degp, b1r, W2)

    g2 = agg(hs2, srcf, dstf, zrows)

    out = pl.pallas_call(
        _tc_final_body, grid=grid,
        in_specs=[g_spec, row_spec, deg_spec, b_spec, row_spec],
        out_specs=row_spec, out_shape=out_sds,
    )(g2, hs2, degp, b2r, concepts)
    return out


# speedup vs baseline: 21.8280x; 2.5034x over previous
import functools

import jax
import jax.numpy as jnp
from jax import lax
from jax.experimental import pallas as pl
from jax.experimental.pallas import tpu as pltpu
from jax.experimental.pallas import tpu_sc as plsc

NC = 2
NS = 16
NW = NC * NS
B = 128
SEG = 40
DW = 128


def _sc_degree(n_pad, n_chunks):
    zr = n_pad // NS
    n_segs = n_chunks // SEG
    mesh = plsc.VectorSubcoreMesh(core_axis_name="c", subcore_axis_name="s")

    @functools.partial(
        pl.kernel,
        mesh=mesh,
        out_type=jax.ShapeDtypeStruct((NC, n_pad, DW), jnp.float32),
        scratch_types=[
            pltpu.VMEM((SEG, B), jnp.int32),
            pltpu.VMEM((B, DW), jnp.float32),
            pltpu.VMEM_SHARED((n_pad, DW), jnp.float32),
        ],
    )
    def deg_kernel(dst_hbm, z_hbm, ones_hbm, out_hbm, idx_v, ones_v, acc):
        c = lax.axis_index("c")
        s = lax.axis_index("s")
        wid = s * NC + c
        pltpu.sync_copy(ones_hbm, ones_v)
        pltpu.sync_copy(z_hbm, acc.at[pl.ds(s * zr, zr)])
        plsc.subcore_barrier()

        def body(j, carry):
            pltpu.sync_copy(ones_v, acc.at[idx_v.at[j]], add=True)
            return carry

        for seg in range(n_segs):
            pltpu.sync_copy(dst_hbm.at[wid, pl.ds(seg * SEG, SEG)], idx_v)
            lax.fori_loop(0, SEG, body, 0)
        plsc.subcore_barrier()
        pltpu.sync_copy(acc.at[pl.ds(s * zr, zr)],
                        out_hbm.at[c, pl.ds(s * zr, zr)])

    return deg_kernel


def _sc_aggregate(n_pad, d, n_chunks):
    zr = n_pad // NS
    n_segs = n_chunks // SEG
    mesh = plsc.VectorSubcoreMesh(core_axis_name="c", subcore_axis_name="s")

    @functools.partial(
        pl.kernel,
        mesh=mesh,
        out_type=jax.ShapeDtypeStruct((NC, n_pad, d), jnp.float32),
        scratch_types=[
            pltpu.VMEM((SEG, B), jnp.int32),
            pltpu.VMEM((SEG, B), jnp.int32),
            pltpu.VMEM((B, d), jnp.float32),
            pltpu.VMEM((B, d), jnp.float32),
            pltpu.VMEM_SHARED((n_pad, d), jnp.float32),
            pltpu.SemaphoreType.DMA,
        ],
    )
    def agg_kernel(hs_hbm, src_hbm, dst_hbm, z_hbm, out_hbm,
                   src_v, dst_v, rows0, rows1, acc, sem):
        c = lax.axis_index("c")
        s = lax.axis_index("s")
        wid = s * NC + c
        pltpu.sync_copy(z_hbm, acc.at[pl.ds(s * zr, zr)])
        plsc.subcore_barrier()

        def body(i, carry):
            j0 = 2 * i
            pltpu.make_async_copy(hs_hbm.at[src_v.at[j0]], rows0, sem).wait()
            pltpu.async_copy(hs_hbm.at[src_v.at[j0 + 1]], rows1, sem)
            pltpu.sync_copy(rows0, acc.at[dst_v.at[j0]], add=True)
            pltpu.make_async_copy(
                hs_hbm.at[src_v.at[j0 + 1]], rows1, sem).wait()

            @pl.when(j0 + 2 < SEG)
            def _():
                pltpu.async_copy(hs_hbm.at[src_v.at[j0 + 2]], rows0, sem)

            pltpu.sync_copy(rows1, acc.at[dst_v.at[j0 + 1]], add=True)
            return carry

        for seg in range(n_segs):
            pltpu.sync_copy(src_hbm.at[wid, pl.ds(seg * SEG, SEG)], src_v)
            pltpu.sync_copy(dst_hbm.at[wid, pl.ds(seg * SEG, SEG)], dst_v)
            pltpu.async_copy(hs_hbm.at[src_v.at[0]], rows0, sem)
            lax.fori_loop(0, SEG // 2, body, 0)
        plsc.subcore_barrier()
        pltpu.sync_copy(acc.at[pl.ds(s * zr, zr)],
                        out_hbm.at[c, pl.ds(s * zr, zr)])

    return agg_kernel


def _dinv_block(degp_ref):
    deg = degp_ref[0][:, 0:1] + degp_ref[1][:, 0:1] + 1.0
    return lax.rsqrt(deg)


def _tc_in_body(x_ref, w_ref, degp_ref, o_ref):
    h = jnp.dot(x_ref[...], w_ref[...], preferred_element_type=jnp.float32)
    o_ref[...] = h * _dinv_block(degp_ref)


def _tc_mid_body(g_ref, hs_ref, degp_ref, b_ref, w_ref, o_ref):
    dinv = _dinv_block(degp_ref)
    x1 = jnp.maximum((g_ref[0] + g_ref[1] + hs_ref[...]) * dinv + b_ref[...],
                     0.0)
    h2 = jnp.dot(x1, w_ref[...], preferred_element_type=jnp.float32)
    o_ref[...] = h2 * dinv


def _tc_final_body(g_ref, hs_ref, degp_ref, b_ref, x0_ref, o_ref):
    dinv = _dinv_block(degp_ref)
    o_ref[...] = (x0_ref[...]
                  + (g_ref[0] + g_ref[1] + hs_ref[...]) * dinv + b_ref[...])


def kernel(concepts, relations, W1, b1, W2, b2):
    n, d = concepts.shape
    e = relations.shape[1]
    assert n % NS == 0, n
    n_pad = ((n + NS * 8) // (NS * 8)) * (NS * 8)
    rel = relations.astype(jnp.int32)

    n_chunks = -(-e // (NW * B * SEG)) * SEG
    pad = n_chunks * NW * B - e
    pad_iota = jnp.arange(pad, dtype=jnp.int32)
    src3 = jnp.concatenate([rel[0], pad_iota % n]).reshape(NW, n_chunks, B)
    dst3 = jnp.concatenate(
        [rel[1], n + pad_iota % (n_pad - n)]).reshape(NW, n_chunks, B)

    ones_rows = jnp.ones((B, DW), jnp.float32)
    zrows = jnp.zeros((n_pad // NS, d), jnp.float32)
    b1r = b1.reshape(1, d)
    b2r = b2.reshape(1, d)

    degp = _sc_degree(n_pad, n_chunks)(dst3, zrows, ones_rows)

    r = 1000 if n % 1000 == 0 else (n // NS)
    grid = (n // r,)
    row_spec = pl.BlockSpec((r, d), lambda i: (i, 0))
    w_spec = pl.BlockSpec((d, d), lambda i: (0, 0))
    deg_spec = pl.BlockSpec((NC, r, DW), lambda i: (0, i, 0))
    g_spec = pl.BlockSpec((NC, r, d), lambda i: (0, i, 0))
    b_spec = pl.BlockSpec((1, d), lambda i: (0, 0))
    out_sds = jax.ShapeDtypeStruct((n, d), jnp.float32)

    hs1 = pl.pallas_call(
        _tc_in_body, grid=grid,
        in_specs=[row_spec, w_spec, deg_spec],
        out_specs=row_spec, out_shape=out_sds,
    )(concepts, W1, degp)

    agg = _sc_aggregate(n_pad, d, n_chunks)
    g1 = agg(hs1, src3, dst3, zrows)

    hs2 = pl.pallas_call(
        _tc_mid_body, grid=grid,
        in_specs=[g_spec, row_spec, deg_spec, b_spec, w_spec],
        out_specs=row_spec, out_shape=out_sds,
    )(g1, hs1, degp, b1r, W2)

    g2 = agg(hs2, src3, dst3, zrows)

    out = pl.pallas_call(
        _tc_final_body, grid=grid,
        in_specs=[g_spec, row_spec, deg_spec, b_spec, row_spec],
        out_specs=row_spec, out_shape=out_sds,
    )(g2, hs2, degp, b2r, concepts)
    return out


# speedup vs baseline: 22.0864x; 1.0118x over previous
import functools

import jax
import jax.numpy as jnp
from jax import lax
from jax.experimental import pallas as pl
from jax.experimental.pallas import tpu as pltpu
from jax.experimental.pallas import tpu_sc as plsc

NC = 2
NS = 16
NW = NC * NS
B = 128
SEG = 40
DW = 128


def _sc_degree(n_pad, n_chunks):
    zr = n_pad // NS
    n_segs = n_chunks // SEG
    mesh = plsc.VectorSubcoreMesh(core_axis_name="c", subcore_axis_name="s")

    @functools.partial(
        pl.kernel,
        mesh=mesh,
        out_type=jax.ShapeDtypeStruct((NC, n_pad, DW), jnp.float32),
        scratch_types=[
            pltpu.VMEM((SEG, B), jnp.int32),
            pltpu.VMEM((B, DW), jnp.float32),
            pltpu.VMEM_SHARED((n_pad, DW), jnp.float32),
        ],
    )
    def deg_kernel(dst_hbm, z_hbm, ones_hbm, out_hbm, idx_v, ones_v, acc):
        c = lax.axis_index("c")
        s = lax.axis_index("s")
        wid = s * NC + c
        pltpu.sync_copy(ones_hbm, ones_v)
        pltpu.sync_copy(z_hbm, acc.at[pl.ds(s * zr, zr)])
        plsc.subcore_barrier()

        def body(j, carry):
            pltpu.sync_copy(ones_v, acc.at[idx_v.at[j]], add=True)
            return carry

        for seg in range(n_segs):
            pltpu.sync_copy(dst_hbm.at[wid, pl.ds(seg * SEG, SEG)], idx_v)
            lax.fori_loop(0, SEG, body, 0)
        plsc.subcore_barrier()
        pltpu.sync_copy(acc.at[pl.ds(s * zr, zr)],
                        out_hbm.at[c, pl.ds(s * zr, zr)])

    return deg_kernel


def _sc_aggregate(n_pad, d, n_chunks):
    zr = n_pad // NS
    n_segs = n_chunks // SEG
    mesh = plsc.VectorSubcoreMesh(core_axis_name="c", subcore_axis_name="s")

    @functools.partial(
        pl.kernel,
        mesh=mesh,
        out_type=jax.ShapeDtypeStruct((NC, n_pad, d), jnp.float32),
        scratch_types=[
            pltpu.VMEM((SEG, B), jnp.int32),
            pltpu.VMEM((SEG, B), jnp.int32),
            pltpu.VMEM((B, d), jnp.float32),
            pltpu.VMEM((B, d), jnp.float32),
            pltpu.VMEM_SHARED((n_pad, d), jnp.float32),
            pltpu.SemaphoreType.DMA,
        ],
    )
    def agg_kernel(hs_hbm, src_hbm, dst_hbm, z_hbm, out_hbm,
                   src_v, dst_v, rows0, rows1, acc, sem):
        c = lax.axis_index("c")
        s = lax.axis_index("s")
        wid = s * NC + c
        pltpu.sync_copy(z_hbm, acc.at[pl.ds(s * zr, zr)])
        plsc.subcore_barrier()

        def body(i, carry):
            j0 = 2 * i
            pltpu.make_async_copy(hs_hbm.at[src_v.at[j0]], rows0, sem).wait()
            pltpu.async_copy(hs_hbm.at[src_v.at[j0 + 1]], rows1, sem)
            pltpu.sync_copy(rows0, acc.at[dst_v.at[j0]], add=True)
            pltpu.make_async_copy(
                hs_hbm.at[src_v.at[j0 + 1]], rows1, sem).wait()

            @pl.when(j0 + 2 < SEG)
            def _():
                pltpu.async_copy(hs_hbm.at[src_v.at[j0 + 2]], rows0, sem)

            pltpu.sync_copy(rows1, acc.at[dst_v.at[j0 + 1]], add=True)
            return carry

        for seg in range(n_segs):
            pltpu.sync_copy(src_hbm.at[wid, pl.ds(seg * SEG, SEG)], src_v)
            pltpu.sync_copy(dst_hbm.at[wid, pl.ds(seg * SEG, SEG)], dst_v)
            pltpu.async_copy(hs_hbm.at[src_v.at[0]], rows0, sem)
            lax.fori_loop(0, SEG // 2, body, 0)
        plsc.subcore_barrier()
        pltpu.sync_copy(acc.at[pl.ds(s * zr, zr)],
                        out_hbm.at[c, pl.ds(s * zr, zr)])

    return agg_kernel


def _dinv_block(degp_ref):
    deg = degp_ref[0][:, 0:1] + degp_ref[1][:, 0:1] + 1.0
    return lax.rsqrt(deg)


def _tc_mm_body(x_ref, w_ref, o_ref):
    o_ref[...] = jnp.dot(x_ref[...], w_ref[...],
                         preferred_element_type=jnp.float32)


def _tc_scale_body(h_ref, degp_ref, o_ref):
    o_ref[...] = h_ref[...] * _dinv_block(degp_ref)


def _tc_mid_body(g_ref, hs_ref, degp_ref, b_ref, w_ref, o_ref):
    dinv = _dinv_block(degp_ref)
    x1 = jnp.maximum((g_ref[0] + g_ref[1] + hs_ref[...]) * dinv + b_ref[...],
                     0.0)
    h2 = jnp.dot(x1, w_ref[...], preferred_element_type=jnp.float32)
    o_ref[...] = h2 * dinv


def _tc_final_body(g_ref, hs_ref, degp_ref, b_ref, x0_ref, o_ref):
    dinv = _dinv_block(degp_ref)
    o_ref[...] = (x0_ref[...]
                  + (g_ref[0] + g_ref[1] + hs_ref[...]) * dinv + b_ref[...])


def kernel(concepts, relations, W1, b1, W2, b2):
    n, d = concepts.shape
    e = relations.shape[1]
    assert n % NS == 0, n
    n_pad = ((n + NS * 8) // (NS * 8)) * (NS * 8)
    rel = relations.astype(jnp.int32)

    n_chunks = -(-e // (NW * B * SEG)) * SEG
    pad = n_chunks * NW * B - e
    pad_iota = jnp.arange(pad, dtype=jnp.int32)
    src3 = jnp.concatenate([rel[0], pad_iota % n]).reshape(NW, n_chunks, B)
    dst3 = jnp.concatenate(
        [rel[1], n + pad_iota % (n_pad - n)]).reshape(NW, n_chunks, B)

    ones_rows = jnp.ones((B, DW), jnp.float32)
    zrows = jnp.zeros((n_pad // NS, d), jnp.float32)
    b1r = b1.reshape(1, d)
    b2r = b2.reshape(1, d)

    degp = _sc_degree(n_pad, n_chunks)(dst3, zrows, ones_rows)

    r = 1000 if n % 1000 == 0 else (n // NS)
    grid = (n // r,)
    row_spec = pl.BlockSpec((r, d), lambda i: (i, 0))
    w_spec = pl.BlockSpec((d, d), lambda i: (0, 0))
    deg_spec = pl.BlockSpec((NC, r, DW), lambda i: (0, i, 0))
    g_spec = pl.BlockSpec((NC, r, d), lambda i: (0, i, 0))
    b_spec = pl.BlockSpec((1, d), lambda i: (0, 0))
    out_sds = jax.ShapeDtypeStruct((n, d), jnp.float32)

    h1 = pl.pallas_call(
        _tc_mm_body, grid=grid,
        in_specs=[row_spec, w_spec],
        out_specs=row_spec, out_shape=out_sds,
    )(concepts, W1)

    hs1 = pl.pallas_call(
        _tc_scale_body, grid=grid,
        in_specs=[row_spec, deg_spec],
        out_specs=row_spec, out_shape=out_sds,
    )(h1, degp)

    agg = _sc_aggregate(n_pad, d, n_chunks)
    g1 = agg(hs1, src3, dst3, zrows)

    hs2 = pl.pallas_call(
        _tc_mid_body, grid=grid,
        in_specs=[g_spec, row_spec, deg_spec, b_spec, w_spec],
        out_specs=row_spec, out_shape=out_sds,
    )(g1, hs1, degp, b1r, W2)

    g2 = agg(hs2, src3, dst3, zrows)

    out = pl.pallas_call(
        _tc_final_body, grid=grid,
        in_specs=[g_spec, row_spec, deg_spec, b_spec, row_spec],
        out_specs=row_spec, out_shape=out_sds,
    )(g2, hs2, degp, b2r, concepts)
    return out
